# Initial kernel scaffold; baseline (speedup 1.0000x reference)
import jax
import jax.numpy as jnp
from jax.experimental import pallas as pl


def kernel(logits, targets):
    raise NotImplementedError("write your pallas kernel here")


# speedup vs baseline: 16.0715x; 16.0715x over previous
import functools

import jax
import jax.numpy as jnp
from jax import lax
from jax.experimental import pallas as pl
from jax.experimental.pallas import tpu as pltpu
from jax.experimental.pallas import tpu_sc as plsc

B = 16
N = 512 * 512
NW = 32
PER_W = (B * N) // NW
CH = 8192
W = 1024
CAP = 8.0
INV_H = W / CAP
SLOTS = W + 1
L = 16


def _sc_hist_kernel(l_hbm, t_hbm, out_hbm, lbuf, tbuf, hist):
    wid = lax.axis_index("s") * 2 + lax.axis_index("c")
    base = wid * PER_W
    lane = lax.iota(jnp.int32, L)

    def zero_body(i, carry):
        hist[pl.ds(i * L, L)] = jnp.zeros((L,), jnp.float32)
        return carry

    lax.fori_loop(0, 2 * SLOTS, zero_body, 0)

    def chunk_body(c, carry):
        off = base + c * CH
        pltpu.sync_copy(l_hbm.at[pl.ds(off, CH)], lbuf)
        pltpu.sync_copy(t_hbm.at[pl.ds(off, CH)], tbuf)

        def vec_body(i, carry2):
            lv = lbuf[pl.ds(i * L, L)]
            tv = tbuf[pl.ds(i * L, L)]
            e = 1.0 - lv * (2.0 * tv - 1.0)
            bi = jnp.minimum((e * INV_H).astype(jnp.int32), W - 1)
            bi = jnp.where(e > 0.0, bi, W)
            addr = bi * L + lane
            plsc.addupdate_scatter(hist, [addr], jnp.ones((L,), jnp.float32))
            plsc.addupdate_scatter(hist, [addr + SLOTS * L], tv)
            return carry2

        lax.fori_loop(0, CH // L, vec_body, 0)
        return carry

    lax.fori_loop(0, PER_W // CH, chunk_body, 0)
    pltpu.sync_copy(hist, out_hbm.at[wid])


def _tc_finish_kernel(hist_ref, out_ref):
    a = hist_ref[...]
    s3 = jnp.sum(a, axis=3)
    rows = jnp.sum(s3, axis=1)
    cv = rows[:, 0:W]
    pv = rows[:, SLOTS:SLOTS + W]
    P = jnp.sum(rows[:, SLOTS:], axis=1, keepdims=True)
    r_iota = lax.broadcasted_iota(jnp.int32, (W, W), 0)
    c_iota = lax.broadcasted_iota(jnp.int32, (W, W), 1)
    tri = (r_iota >= c_iota).astype(jnp.float32)
    S = lax.dot_general(cv, tri, (((1,), (0,)), ((), ())),
                        preferred_element_type=jnp.float32)
    Sp = lax.dot_general(pv, tri, (((1,), (0,)), ((), ())),
                         preferred_element_type=jnp.float32)
    J = 1.0 - (P - Sp) / jnp.maximum(P + S - Sp, 1.0)
    J_top = 1.0 - P / jnp.maximum(P, 1.0)
    h = CAP / W
    row_sum = h * (jnp.sum(J[:, 1:], axis=1, keepdims=True)
                   + 0.5 * (J[:, 0:1] + J_top))
    loss = jnp.sum(row_sum) / (B * N)
    out_ref[...] = jnp.full((8, 128), loss, jnp.float32)


def kernel(logits, targets):
    lflat = logits.reshape(B * N)
    tflat = targets.reshape(B * N)

    mesh = plsc.VectorSubcoreMesh(core_axis_name="c", subcore_axis_name="s")
    sc_hist = functools.partial(
        pl.kernel,
        mesh=mesh,
        compiler_params=pltpu.CompilerParams(needs_layout_passes=False),
        out_type=jax.ShapeDtypeStruct((NW, 2 * SLOTS * L), jnp.float32),
        scratch_types=[
            pltpu.VMEM((CH,), jnp.float32),
            pltpu.VMEM((CH,), jnp.float32),
            pltpu.VMEM((2 * SLOTS * L,), jnp.float32),
        ],
    )(_sc_hist_kernel)

    hist = sc_hist(lflat, tflat)
    hist4 = hist.reshape(B, 2, 2 * SLOTS, L)

    res = pl.pallas_call(
        _tc_finish_kernel,
        out_shape=jax.ShapeDtypeStruct((8, 128), jnp.float32),
    )(hist4)
    return res[0, 0]


# speedup vs baseline: 18.8915x; 1.1755x over previous
import functools

import jax
import jax.numpy as jnp
from jax import lax
from jax.experimental import pallas as pl
from jax.experimental.pallas import tpu as pltpu
from jax.experimental.pallas import tpu_sc as plsc

B = 16
N = 512 * 512
NW = 32
PER_W = (B * N) // NW
CH = 8192
W = 1024
CAP = 8.0
INV_H = W / CAP
SLOTS = W + 1
L = 16
HL = 2 * SLOTS
UNROLL = 8


def _sc_hist_kernel(l_hbm, t_hbm, out_hbm, lbuf, tbuf, hist):
    wid = lax.axis_index("s") * 2 + lax.axis_index("c")
    base = wid * PER_W
    lane_base = lax.iota(jnp.int32, L) * HL
    ones = jnp.ones((L,), jnp.float32)
    zeros = jnp.zeros((L,), jnp.float32)

    def zero_body(i, carry):
        for u in range(10):
            hist[pl.ds((i * 10 + u) * L, L)] = zeros
        return carry

    lax.fori_loop(0, HL // 10, zero_body, 0, unroll=False)

    def chunk_body(c, carry):
        off = base + c * CH
        pltpu.sync_copy(l_hbm.at[pl.ds(off, CH)], lbuf)
        pltpu.sync_copy(t_hbm.at[pl.ds(off, CH)], tbuf)

        def vec_body(i, carry2):
            for u in range(UNROLL):
                o = (i * UNROLL + u) * L
                lv = lbuf[pl.ds(o, L)]
                tv = tbuf[pl.ds(o, L)]
                e = 1.0 - lv * (2.0 * tv - 1.0)
                bi = jnp.minimum((e * INV_H).astype(jnp.int32), W - 1)
                bi = jnp.where(e > 0.0, bi, W)
                addr = lane_base + tv.astype(jnp.int32) * SLOTS + bi
                plsc.addupdate_scatter(hist, [addr], ones)
            return carry2

        lax.fori_loop(0, CH // (L * UNROLL), vec_body, 0, unroll=False)
        return carry

    lax.fori_loop(0, PER_W // CH, chunk_body, 0, unroll=False)
    pltpu.sync_copy(hist, out_hbm.at[wid])


def _tc_finish_kernel(hist_ref, out_ref):
    a = hist_ref[...]
    s2 = jnp.sum(a, axis=2)
    rows = jnp.sum(s2, axis=1)
    nv = rows[:, 0:W]
    pv = rows[:, SLOTS:SLOTS + W]
    cv = nv + pv
    P = jnp.sum(rows[:, SLOTS:], axis=1, keepdims=True)
    r_iota = lax.broadcasted_iota(jnp.int32, (W, W), 0)
    c_iota = lax.broadcasted_iota(jnp.int32, (W, W), 1)
    tri = (r_iota >= c_iota).astype(jnp.float32)
    S = lax.dot_general(cv, tri, (((1,), (0,)), ((), ())),
                        preferred_element_type=jnp.float32)
    Sp = lax.dot_general(pv, tri, (((1,), (0,)), ((), ())),
                         preferred_element_type=jnp.float32)
    J = 1.0 - (P - Sp) / jnp.maximum(P + S - Sp, 1.0)
    J_top = 1.0 - P / jnp.maximum(P, 1.0)
    h = CAP / W
    row_sum = h * (jnp.sum(J[:, 1:], axis=1, keepdims=True)
                   + 0.5 * (J[:, 0:1] + J_top))
    loss = jnp.sum(row_sum) / (B * N)
    out_ref[...] = jnp.full((8, 128), loss, jnp.float32)


def kernel(logits, targets):
    lflat = logits.reshape(B * N)
    tflat = targets.reshape(B * N)

    mesh = plsc.VectorSubcoreMesh(core_axis_name="c", subcore_axis_name="s")
    sc_hist = functools.partial(
        pl.kernel,
        mesh=mesh,
        compiler_params=pltpu.CompilerParams(needs_layout_passes=False),
        out_type=jax.ShapeDtypeStruct((NW, L * HL), jnp.float32),
        scratch_types=[
            pltpu.VMEM((CH,), jnp.float32),
            pltpu.VMEM((CH,), jnp.float32),
            pltpu.VMEM((L * HL,), jnp.float32),
        ],
    )(_sc_hist_kernel)

    hist = sc_hist(lflat, tflat)
    hist4 = hist.reshape(B, 2, L, HL)

    res = pl.pallas_call(
        _tc_finish_kernel,
        out_shape=jax.ShapeDtypeStruct((8, 128), jnp.float32),
    )(hist4)
    return res[0, 0]
